# baseline (speedup 1.0000x reference)
import functools

import jax
import jax.numpy as jnp
from jax import lax
from jax.experimental import pallas as pl
from jax.experimental.pallas import tpu as pltpu
from jax.experimental.pallas import tpu_sc as plsc

BATCH = 16384
SEQ = 200
EMBED = 32
HIDDEN = 128
CLS = 10

NC, NS = 2, 16
NW = NC * NS
ROWS_PER_W = BATCH // NW
HALF = 256
SEQ_A = 128
SEQ_B = SEQ - SEQ_A
NBUF = 4
INV_SEQ = 1.0 / SEQ


VOCAB = 1000000
TCOLS = 800
TBLOCKS = VOCAB // TCOLS


def _transpose_sc(tableT):
    mesh = plsc.VectorSubcoreMesh(core_axis_name="c", subcore_axis_name="s")

    @functools.partial(
        pl.kernel,
        out_type=jax.ShapeDtypeStruct((VOCAB, EMBED), jnp.float32),
        mesh=mesh,
        compiler_params=pltpu.CompilerParams(
            use_tc_tiling_on_sc=False, needs_layout_passes=False),
    )
    def k(t_hbm, out_hbm):
        def body(in_v, out_v):
            lanes = jax.lax.iota(jnp.int32, 16)

            @pl.loop(0, TCOLS, step=4)
            def _c(c):
                for u in range(4):
                    cc = c + u
                    cols = jnp.full((16,), 0, jnp.int32) + cc
                    for h in range(2):
                        rows = lanes + (16 * h)
                        g = plsc.load_gather(in_v, [rows, cols])
                        out_v[cc, pl.ds(16 * h, 16)] = g

        pltpu.emit_pipeline(
            body,
            grid=(TBLOCKS,),
            in_specs=[pl.BlockSpec((EMBED, TCOLS), lambda i: (0, i))],
            out_specs=[pl.BlockSpec((TCOLS, EMBED), lambda i: (i, 0))],
            core_axis_name=("c", "s"),
            dimension_semantics=(pltpu.PARALLEL,),
        )(t_hbm, out_hbm)

    return k(tableT)


def _pool_sc(x, table):
    mesh = plsc.VectorSubcoreMesh(core_axis_name="c", subcore_axis_name="s")

    @functools.partial(
        pl.kernel,
        out_type=jax.ShapeDtypeStruct((BATCH, EMBED), jnp.float32),
        mesh=mesh,
        scratch_types=[
            pltpu.VMEM((HALF, SEQ), jnp.int32),
            pltpu.VMEM((NBUF, SEQ, EMBED), jnp.float32),
            pltpu.VMEM((HALF, EMBED), jnp.float32),
            [pltpu.SemaphoreType.DMA] * NBUF,
        ],
        compiler_params=pltpu.CompilerParams(use_tc_tiling_on_sc=False),
    )
    def k(x_hbm, tab_hbm, out_hbm, idx_v, g_v, o_v, sems):
        wid = lax.axis_index("s") * NC + lax.axis_index("c")
        base = wid * ROWS_PER_W

        def issue(row, b):
            pltpu.async_copy(
                tab_hbm.at[idx_v.at[row, pl.ds(0, SEQ_A)]],
                g_v.at[b, pl.ds(0, SEQ_A)], sems[b])
            pltpu.async_copy(
                tab_hbm.at[idx_v.at[row, pl.ds(SEQ_A, SEQ_B)]],
                g_v.at[b, pl.ds(SEQ_A, SEQ_B)], sems[b])

        def drain(b):
            pltpu.make_async_copy(
                tab_hbm.at[idx_v.at[0, pl.ds(0, SEQ_A)]],
                g_v.at[b, pl.ds(0, SEQ_A)], sems[b]).wait()
            pltpu.make_async_copy(
                tab_hbm.at[idx_v.at[0, pl.ds(SEQ_A, SEQ_B)]],
                g_v.at[b, pl.ds(SEQ_A, SEQ_B)], sems[b]).wait()

        for half in range(ROWS_PER_W // HALF):
            hbase = base + half * HALF
            pltpu.sync_copy(x_hbm.at[pl.ds(hbase, HALF)], idx_v)
            for b in range(NBUF):
                issue(b, b)

            @pl.loop(0, HALF, step=NBUF)
            def _rows(rc):
                for b in range(NBUF):
                    r = rc + b
                    drain(b)

                    def body(i, carry):
                        a0, a1 = carry
                        return (a0 + g_v[b, i, pl.ds(0, 16)],
                                a1 + g_v[b, i, pl.ds(16, 16)])

                    a0, a1 = lax.fori_loop(
                        0, SEQ, body,
                        (jnp.zeros((16,), jnp.float32),
                         jnp.zeros((16,), jnp.float32)),
                        unroll=8)
                    o_v[r, pl.ds(0, 16)] = a0 * INV_SEQ
                    o_v[r, pl.ds(16, 16)] = a1 * INV_SEQ

                    @pl.when(rc + NBUF < HALF)
                    def _prefetch():
                        issue(r + NBUF, b)

            pltpu.sync_copy(o_v, out_hbm.at[pl.ds(hbase, HALF)])

    return k(x, table)


def _mlp_tc(pooled, W1, b1, W2, b2):
    BB = 2048

    def body(p_ref, w1_ref, b1_ref, w2_ref, b2_ref, o_ref):
        h = jnp.dot(p_ref[...], w1_ref[...],
                    preferred_element_type=jnp.float32)
        h = jnp.maximum(h + b1_ref[...], 0.0)
        o_ref[...] = jnp.dot(h, w2_ref[...],
                             preferred_element_type=jnp.float32) + b2_ref[...]

    return pl.pallas_call(
        body,
        grid=(BATCH // BB,),
        in_specs=[
            pl.BlockSpec((BB, EMBED), lambda i: (i, 0)),
            pl.BlockSpec((EMBED, HIDDEN), lambda i: (0, 0)),
            pl.BlockSpec((1, HIDDEN), lambda i: (0, 0)),
            pl.BlockSpec((HIDDEN, CLS), lambda i: (0, 0)),
            pl.BlockSpec((1, CLS), lambda i: (0, 0)),
        ],
        out_specs=pl.BlockSpec((BB, CLS), lambda i: (i, 0)),
        out_shape=jax.ShapeDtypeStruct((BATCH, CLS), jnp.float32),
    )(pooled, W1, b1.reshape(1, HIDDEN), W2, b2.reshape(1, CLS))


def kernel(x, table, W1, b1, W2, b2):
    table_rm = _transpose_sc(table.T)
    pooled = _pool_sc(x, table_rm)
    return _mlp_tc(pooled, W1, b1, W2, b2)


# speedup vs baseline: 4.1363x; 4.1363x over previous
import functools

import jax
import jax.numpy as jnp
from jax import lax
from jax.experimental import pallas as pl
from jax.experimental.pallas import tpu as pltpu
from jax.experimental.pallas import tpu_sc as plsc

BATCH = 16384
SEQ = 200
EMBED = 32
HIDDEN = 128
CLS = 10

NC, NS = 2, 16
NW = NC * NS
ROWS_PER_W = BATCH // NW
HALF = 256
SEQ_A = 128
SEQ_B = SEQ - SEQ_A
NBUF = 4
INV_SEQ = 1.0 / SEQ


VOCAB = 1000000
CTILES = VOCAB // 128
CREM = VOCAB - CTILES * 128


def _transpose_sc(tableT, tail2d):
    mesh = plsc.VectorSubcoreMesh(core_axis_name="c", subcore_axis_name="s")

    @functools.partial(
        pl.kernel,
        out_type=jax.ShapeDtypeStruct((VOCAB * EMBED // 128, 128),
                                      jnp.float32),
        mesh=mesh,
        scratch_types=[
            [pltpu.VMEM((4, 8, 128), jnp.float32)] * 2,
            [pltpu.VMEM((32, 128), jnp.float32)] * 2,
            [pltpu.SemaphoreType.DMA] * 2,
            [pltpu.SemaphoreType.DMA] * 2,
        ],
        compiler_params=pltpu.CompilerParams(
            use_tc_tiling_on_sc=True, needs_layout_passes=False),
    )
    def k(t_hbm, tail_hbm, flat_hbm, in_v, out_v, in_sems, out_sems):
        wid = lax.axis_index("s") * NC + lax.axis_index("c")
        lanes = jax.lax.iota(jnp.int32, 16)
        rbase = lanes // 4
        cbase = (lanes % 4) * 32

        def fire_in(ct, b):
            for fb in range(4):
                pltpu.async_copy(
                    t_hbm.at[pl.ds(8 * fb, 8), pl.ds(ct * 128, 128)],
                    in_v[b].at[fb], in_sems[b])

        def drain_in(b):
            for fb in range(4):
                pltpu.make_async_copy(
                    t_hbm.at[pl.ds(0, 8), pl.ds(0, 128)],
                    in_v[b].at[fb], in_sems[b]).wait()

        def drain_out(b):
            pltpu.make_async_copy(
                out_v[b], flat_hbm.at[pl.ds(0, 32)],
                out_sems[b]).wait()

        def fold(b, ncols):
            for f in range(32):
                fb, fi = f // 8, f % 8
                cols = cbase + f
                for u in range(ncols // 16):
                    v = in_v[b][fb, fi, pl.ds(16 * u, 16)]
                    plsc.store_scatter(
                        out_v[b], [rbase + 4 * u, cols], v)

        for b in range(2):
            @pl.when(wid + 32 * b < CTILES)
            def _prime():
                fire_in(wid + 32 * b, b)

        nj = CTILES // NW + 2
        @pl.loop(0, nj, step=2)
        def _blocks(jj):
            for b in range(2):
                j = jj + b
                ct = wid + 32 * j

                @pl.when(ct < CTILES)
                def _one():
                    drain_in(b)

                    @pl.when(j >= 2)
                    def _w():
                        drain_out(b)

                    fold(b, 128)
                    pltpu.async_copy(
                        out_v[b],
                        flat_hbm.at[pl.ds(ct * 32, 32)], out_sems[b])

                    @pl.when(ct + 2 * NW < CTILES)
                    def _next():
                        fire_in(ct + 2 * NW, b)

        for b in range(2):
            drain_out(b)

        @pl.when(wid == NW - 1)
        def _tail():
            pltpu.sync_copy(tail_hbm, out_v[0].at[pl.ds(0, 16)])
            pltpu.sync_copy(out_v[0].at[pl.ds(0, 16)],
                            flat_hbm.at[pl.ds(CTILES * 32, 16)])

    return k(tableT, tail2d)


def _pool_sc(x, table):
    mesh = plsc.VectorSubcoreMesh(core_axis_name="c", subcore_axis_name="s")

    @functools.partial(
        pl.kernel,
        out_type=jax.ShapeDtypeStruct((BATCH, EMBED), jnp.float32),
        mesh=mesh,
        scratch_types=[
            pltpu.VMEM((HALF, SEQ), jnp.int32),
            pltpu.VMEM((NBUF, SEQ, EMBED), jnp.float32),
            pltpu.VMEM((HALF, EMBED), jnp.float32),
            [pltpu.SemaphoreType.DMA] * NBUF,
        ],
        compiler_params=pltpu.CompilerParams(use_tc_tiling_on_sc=False),
    )
    def k(x_hbm, tab_hbm, out_hbm, idx_v, g_v, o_v, sems):
        wid = lax.axis_index("s") * NC + lax.axis_index("c")
        base = wid * ROWS_PER_W

        def issue(row, b):
            pltpu.async_copy(
                tab_hbm.at[idx_v.at[row, pl.ds(0, SEQ_A)]],
                g_v.at[b, pl.ds(0, SEQ_A)], sems[b])
            pltpu.async_copy(
                tab_hbm.at[idx_v.at[row, pl.ds(SEQ_A, SEQ_B)]],
                g_v.at[b, pl.ds(SEQ_A, SEQ_B)], sems[b])

        def drain(b):
            pltpu.make_async_copy(
                tab_hbm.at[idx_v.at[0, pl.ds(0, SEQ_A)]],
                g_v.at[b, pl.ds(0, SEQ_A)], sems[b]).wait()
            pltpu.make_async_copy(
                tab_hbm.at[idx_v.at[0, pl.ds(SEQ_A, SEQ_B)]],
                g_v.at[b, pl.ds(SEQ_A, SEQ_B)], sems[b]).wait()

        for half in range(ROWS_PER_W // HALF):
            hbase = base + half * HALF
            pltpu.sync_copy(x_hbm.at[pl.ds(hbase, HALF)], idx_v)
            for b in range(NBUF):
                issue(b, b)

            @pl.loop(0, HALF, step=NBUF)
            def _rows(rc):
                for b in range(NBUF):
                    r = rc + b
                    drain(b)

                    def body(i, carry):
                        a0, a1 = carry
                        return (a0 + g_v[b, i, pl.ds(0, 16)],
                                a1 + g_v[b, i, pl.ds(16, 16)])

                    a0, a1 = lax.fori_loop(
                        0, SEQ, body,
                        (jnp.zeros((16,), jnp.float32),
                         jnp.zeros((16,), jnp.float32)),
                        unroll=8)
                    o_v[r, pl.ds(0, 16)] = a0 * INV_SEQ
                    o_v[r, pl.ds(16, 16)] = a1 * INV_SEQ

                    @pl.when(rc + NBUF < HALF)
                    def _prefetch():
                        issue(r + NBUF, b)

            pltpu.sync_copy(o_v, out_hbm.at[pl.ds(hbase, HALF)])

    return k(x, table)


def _mlp_tc(pooled, W1, b1, W2, b2):
    BB = 2048

    def body(p_ref, w1_ref, b1_ref, w2_ref, b2_ref, o_ref):
        h = jnp.dot(p_ref[...], w1_ref[...],
                    preferred_element_type=jnp.float32)
        h = jnp.maximum(h + b1_ref[...], 0.0)
        o_ref[...] = jnp.dot(h, w2_ref[...],
                             preferred_element_type=jnp.float32) + b2_ref[...]

    return pl.pallas_call(
        body,
        grid=(BATCH // BB,),
        in_specs=[
            pl.BlockSpec((BB, EMBED), lambda i: (i, 0)),
            pl.BlockSpec((EMBED, HIDDEN), lambda i: (0, 0)),
            pl.BlockSpec((1, HIDDEN), lambda i: (0, 0)),
            pl.BlockSpec((HIDDEN, CLS), lambda i: (0, 0)),
            pl.BlockSpec((1, CLS), lambda i: (0, 0)),
        ],
        out_specs=pl.BlockSpec((BB, CLS), lambda i: (i, 0)),
        out_shape=jax.ShapeDtypeStruct((BATCH, CLS), jnp.float32),
    )(pooled, W1, b1.reshape(1, HIDDEN), W2, b2.reshape(1, CLS))


def kernel(x, table, W1, b1, W2, b2):
    tail2d = table[CTILES * 128:].reshape(16, 128)
    table_rm = _transpose_sc(table.T, tail2d).reshape(VOCAB, EMBED)
    pooled = _pool_sc(x, table_rm)
    return _mlp_tc(pooled, W1, b1, W2, b2)


# speedup vs baseline: 4.1528x; 1.0040x over previous
import functools

import jax
import jax.numpy as jnp
from jax import lax
from jax.experimental import pallas as pl
from jax.experimental.pallas import tpu as pltpu
from jax.experimental.pallas import tpu_sc as plsc

BATCH = 16384
SEQ = 200
EMBED = 32
HIDDEN = 128
CLS = 10

NC, NS = 2, 16
NW = NC * NS
ROWS_PER_W = BATCH // NW
HALF = 256
SEQ_A = 128
SEQ_B = SEQ - SEQ_A
NBUF = 4
INV_SEQ = 1.0 / SEQ


VOCAB = 1000000
CTILES = VOCAB // 128
CREM = VOCAB - CTILES * 128


def _transpose_sc(tableT, tail2d):
    mesh = plsc.VectorSubcoreMesh(core_axis_name="c", subcore_axis_name="s")

    @functools.partial(
        pl.kernel,
        out_type=jax.ShapeDtypeStruct((VOCAB * EMBED // 128, 128),
                                      jnp.float32),
        mesh=mesh,
        scratch_types=[
            [pltpu.VMEM((4, 8, 128), jnp.float32)] * 4,
            [pltpu.VMEM((32, 128), jnp.float32)] * 4,
            [pltpu.SemaphoreType.DMA] * 4,
            [pltpu.SemaphoreType.DMA] * 4,
        ],
        compiler_params=pltpu.CompilerParams(
            use_tc_tiling_on_sc=True, needs_layout_passes=False),
    )
    def k(t_hbm, tail_hbm, flat_hbm, in_v, out_v, in_sems, out_sems):
        wid = lax.axis_index("s") * NC + lax.axis_index("c")
        lanes = jax.lax.iota(jnp.int32, 16)
        rbase = lanes // 4
        cbase = (lanes % 4) * 32

        def fire_in(ct, b):
            for fb in range(4):
                pltpu.async_copy(
                    t_hbm.at[pl.ds(8 * fb, 8), pl.ds(ct * 128, 128)],
                    in_v[b].at[fb], in_sems[b])

        def drain_in(b):
            for fb in range(4):
                pltpu.make_async_copy(
                    t_hbm.at[pl.ds(0, 8), pl.ds(0, 128)],
                    in_v[b].at[fb], in_sems[b]).wait()

        def drain_out(b):
            pltpu.make_async_copy(
                out_v[b], flat_hbm.at[pl.ds(0, 32)],
                out_sems[b]).wait()

        def fold(b):
            rows = [rbase + 4 * u for u in range(8)]

            @pl.loop(0, 32)
            def _f(f):
                fb, fi = f // 8, f % 8
                cols = cbase + f
                vs = [in_v[b][fb, fi, pl.ds(16 * u, 16)] for u in range(8)]
                for u in range(8):
                    plsc.store_scatter(out_v[b], [rows[u], cols], vs[u])

        RING = 4
        for b in range(RING):
            @pl.when(wid + 32 * b < CTILES)
            def _prime():
                fire_in(wid + 32 * b, b)

        nj = CTILES // NW + RING
        @pl.loop(0, nj, step=RING)
        def _blocks(jj):
            for b in range(RING):
                j = jj + b
                ct = wid + 32 * j

                @pl.when(ct < CTILES)
                def _one():
                    drain_in(b)

                    @pl.when(j >= RING)
                    def _w():
                        drain_out(b)

                    fold(b)

                    @pl.when(ct + RING * NW < CTILES)
                    def _next():
                        fire_in(ct + RING * NW, b)

                    pltpu.async_copy(
                        out_v[b],
                        flat_hbm.at[pl.ds(ct * 32, 32)], out_sems[b])

        for b in range(RING):
            drain_out(b)

        @pl.when(wid == NW - 1)
        def _tail():
            pltpu.sync_copy(tail_hbm, out_v[0].at[pl.ds(0, 16)])
            pltpu.sync_copy(out_v[0].at[pl.ds(0, 16)],
                            flat_hbm.at[pl.ds(CTILES * 32, 16)])

    return k(tableT, tail2d)


def _pool_sc(x, table):
    mesh = plsc.VectorSubcoreMesh(core_axis_name="c", subcore_axis_name="s")

    @functools.partial(
        pl.kernel,
        out_type=jax.ShapeDtypeStruct((BATCH, EMBED), jnp.float32),
        mesh=mesh,
        scratch_types=[
            pltpu.VMEM((HALF, SEQ), jnp.int32),
            pltpu.VMEM((NBUF, SEQ, EMBED), jnp.float32),
            pltpu.VMEM((HALF, EMBED), jnp.float32),
            [pltpu.SemaphoreType.DMA] * NBUF,
        ],
        compiler_params=pltpu.CompilerParams(use_tc_tiling_on_sc=False),
    )
    def k(x_hbm, tab_hbm, out_hbm, idx_v, g_v, o_v, sems):
        wid = lax.axis_index("s") * NC + lax.axis_index("c")
        base = wid * ROWS_PER_W

        def issue(row, b):
            pltpu.async_copy(
                tab_hbm.at[idx_v.at[row, pl.ds(0, SEQ_A)]],
                g_v.at[b, pl.ds(0, SEQ_A)], sems[b])
            pltpu.async_copy(
                tab_hbm.at[idx_v.at[row, pl.ds(SEQ_A, SEQ_B)]],
                g_v.at[b, pl.ds(SEQ_A, SEQ_B)], sems[b])

        def drain(b):
            pltpu.make_async_copy(
                tab_hbm.at[idx_v.at[0, pl.ds(0, SEQ_A)]],
                g_v.at[b, pl.ds(0, SEQ_A)], sems[b]).wait()
            pltpu.make_async_copy(
                tab_hbm.at[idx_v.at[0, pl.ds(SEQ_A, SEQ_B)]],
                g_v.at[b, pl.ds(SEQ_A, SEQ_B)], sems[b]).wait()

        for half in range(ROWS_PER_W // HALF):
            hbase = base + half * HALF
            pltpu.sync_copy(x_hbm.at[pl.ds(hbase, HALF)], idx_v)
            for b in range(NBUF):
                issue(b, b)

            @pl.loop(0, HALF, step=NBUF)
            def _rows(rc):
                for b in range(NBUF):
                    r = rc + b
                    drain(b)

                    def body(i, carry):
                        a0, a1 = carry
                        return (a0 + g_v[b, i, pl.ds(0, 16)],
                                a1 + g_v[b, i, pl.ds(16, 16)])

                    a0, a1 = lax.fori_loop(
                        0, SEQ, body,
                        (jnp.zeros((16,), jnp.float32),
                         jnp.zeros((16,), jnp.float32)),
                        unroll=8)
                    o_v[r, pl.ds(0, 16)] = a0 * INV_SEQ
                    o_v[r, pl.ds(16, 16)] = a1 * INV_SEQ

                    @pl.when(rc + NBUF < HALF)
                    def _prefetch():
                        issue(r + NBUF, b)

            pltpu.sync_copy(o_v, out_hbm.at[pl.ds(hbase, HALF)])

    return k(x, table)


def _mlp_tc(pooled, W1, b1, W2, b2):
    BB = 2048

    def body(p_ref, w1_ref, b1_ref, w2_ref, b2_ref, o_ref):
        h = jnp.dot(p_ref[...], w1_ref[...],
                    preferred_element_type=jnp.float32)
        h = jnp.maximum(h + b1_ref[...], 0.0)
        o_ref[...] = jnp.dot(h, w2_ref[...],
                             preferred_element_type=jnp.float32) + b2_ref[...]

    return pl.pallas_call(
        body,
        grid=(BATCH // BB,),
        in_specs=[
            pl.BlockSpec((BB, EMBED), lambda i: (i, 0)),
            pl.BlockSpec((EMBED, HIDDEN), lambda i: (0, 0)),
            pl.BlockSpec((1, HIDDEN), lambda i: (0, 0)),
            pl.BlockSpec((HIDDEN, CLS), lambda i: (0, 0)),
            pl.BlockSpec((1, CLS), lambda i: (0, 0)),
        ],
        out_specs=pl.BlockSpec((BB, CLS), lambda i: (i, 0)),
        out_shape=jax.ShapeDtypeStruct((BATCH, CLS), jnp.float32),
    )(pooled, W1, b1.reshape(1, HIDDEN), W2, b2.reshape(1, CLS))


def kernel(x, table, W1, b1, W2, b2):
    tail2d = table[CTILES * 128:].reshape(16, 128)
    table_rm = _transpose_sc(table.T, tail2d).reshape(VOCAB, EMBED)
    pooled = _pool_sc(x, table_rm)
    return _mlp_tc(pooled, W1, b1, W2, b2)


# speedup vs baseline: 4.1806x; 1.0067x over previous
import functools

import jax
import jax.numpy as jnp
from jax import lax
from jax.experimental import pallas as pl
from jax.experimental.pallas import tpu as pltpu
from jax.experimental.pallas import tpu_sc as plsc

BATCH = 16384
SEQ = 200
EMBED = 32
HIDDEN = 128
CLS = 10

NC, NS = 2, 16
NW = NC * NS
ROWS_PER_W = BATCH // NW
HALF = 256
SEQ_A = 128
SEQ_B = SEQ - SEQ_A
NBUF = 4
INV_SEQ = 1.0 / SEQ


VOCAB = 1000000
CTILES = VOCAB // 128
CREM = VOCAB - CTILES * 128


def _transpose_sc(tableT, tail2d):
    mesh = plsc.VectorSubcoreMesh(core_axis_name="c", subcore_axis_name="s")

    @functools.partial(
        pl.kernel,
        out_type=jax.ShapeDtypeStruct((VOCAB * EMBED // 128, 128),
                                      jnp.float32),
        mesh=mesh,
        scratch_types=[
            [pltpu.VMEM((32, 128), jnp.float32)] * 6,
            [pltpu.VMEM((32, 128), jnp.float32)] * 6,
            [pltpu.SemaphoreType.DMA] * 6,
            [pltpu.SemaphoreType.DMA] * 6,
        ],
        compiler_params=pltpu.CompilerParams(
            use_tc_tiling_on_sc=True, needs_layout_passes=False),
    )
    def k(t_hbm, tail_hbm, flat_hbm, in_v, out_v, in_sems, out_sems):
        wid = lax.axis_index("s") * NC + lax.axis_index("c")
        lanes = jax.lax.iota(jnp.int32, 16)
        rbase = lanes // 4
        cbase = (lanes % 4) * 32

        def fire_in(ct, b):
            pltpu.async_copy(
                t_hbm.at[:, pl.ds(ct * 128, 128)], in_v[b], in_sems[b])

        def drain_in(b):
            pltpu.make_async_copy(
                t_hbm.at[:, pl.ds(0, 128)], in_v[b], in_sems[b]).wait()

        def drain_out(b):
            pltpu.make_async_copy(
                out_v[b], flat_hbm.at[pl.ds(0, 32)],
                out_sems[b]).wait()

        def fold(b):
            rows = [rbase + 4 * u for u in range(8)]

            @pl.loop(0, 32)
            def _f(f):
                cols = cbase + f
                vs = [in_v[b][f, pl.ds(16 * u, 16)] for u in range(8)]
                for u in range(8):
                    plsc.store_scatter(out_v[b], [rows[u], cols], vs[u])

        RING = 6
        for b in range(RING):
            @pl.when(wid + 32 * b < CTILES)
            def _prime():
                fire_in(wid + 32 * b, b)

        nj = CTILES // NW + RING
        @pl.loop(0, nj, step=RING)
        def _blocks(jj):
            for b in range(RING):
                j = jj + b
                ct = wid + 32 * j

                @pl.when(ct < CTILES)
                def _one():
                    drain_in(b)

                    @pl.when(j >= RING)
                    def _w():
                        drain_out(b)

                    fold(b)

                    @pl.when(ct + RING * NW < CTILES)
                    def _next():
                        fire_in(ct + RING * NW, b)

                    pltpu.async_copy(
                        out_v[b],
                        flat_hbm.at[pl.ds(ct * 32, 32)], out_sems[b])

        for b in range(RING):
            drain_out(b)

        @pl.when(wid == NW - 1)
        def _tail():
            pltpu.sync_copy(tail_hbm, out_v[0].at[pl.ds(0, 16)])
            pltpu.sync_copy(out_v[0].at[pl.ds(0, 16)],
                            flat_hbm.at[pl.ds(CTILES * 32, 16)])

    return k(tableT, tail2d)


def _pool_sc(x, table):
    mesh = plsc.VectorSubcoreMesh(core_axis_name="c", subcore_axis_name="s")

    @functools.partial(
        pl.kernel,
        out_type=jax.ShapeDtypeStruct((BATCH, EMBED), jnp.float32),
        mesh=mesh,
        scratch_types=[
            pltpu.VMEM((HALF, SEQ), jnp.int32),
            pltpu.VMEM((NBUF, SEQ, EMBED), jnp.float32),
            pltpu.VMEM((HALF, EMBED), jnp.float32),
            [pltpu.SemaphoreType.DMA] * NBUF,
        ],
        compiler_params=pltpu.CompilerParams(use_tc_tiling_on_sc=False),
    )
    def k(x_hbm, tab_hbm, out_hbm, idx_v, g_v, o_v, sems):
        wid = lax.axis_index("s") * NC + lax.axis_index("c")
        base = wid * ROWS_PER_W

        def issue(row, b):
            pltpu.async_copy(
                tab_hbm.at[idx_v.at[row, pl.ds(0, SEQ_A)]],
                g_v.at[b, pl.ds(0, SEQ_A)], sems[b])
            pltpu.async_copy(
                tab_hbm.at[idx_v.at[row, pl.ds(SEQ_A, SEQ_B)]],
                g_v.at[b, pl.ds(SEQ_A, SEQ_B)], sems[b])

        def drain(b):
            pltpu.make_async_copy(
                tab_hbm.at[idx_v.at[0, pl.ds(0, SEQ_A)]],
                g_v.at[b, pl.ds(0, SEQ_A)], sems[b]).wait()
            pltpu.make_async_copy(
                tab_hbm.at[idx_v.at[0, pl.ds(SEQ_A, SEQ_B)]],
                g_v.at[b, pl.ds(SEQ_A, SEQ_B)], sems[b]).wait()

        for half in range(ROWS_PER_W // HALF):
            hbase = base + half * HALF
            pltpu.sync_copy(x_hbm.at[pl.ds(hbase, HALF)], idx_v)
            for b in range(NBUF):
                issue(b, b)

            @pl.loop(0, HALF, step=NBUF)
            def _rows(rc):
                for b in range(NBUF):
                    r = rc + b
                    drain(b)

                    def body(i, carry):
                        a0, a1 = carry
                        return (a0 + g_v[b, i, pl.ds(0, 16)],
                                a1 + g_v[b, i, pl.ds(16, 16)])

                    a0, a1 = lax.fori_loop(
                        0, SEQ, body,
                        (jnp.zeros((16,), jnp.float32),
                         jnp.zeros((16,), jnp.float32)),
                        unroll=8)
                    o_v[r, pl.ds(0, 16)] = a0 * INV_SEQ
                    o_v[r, pl.ds(16, 16)] = a1 * INV_SEQ

                    @pl.when(rc + NBUF < HALF)
                    def _prefetch():
                        issue(r + NBUF, b)

            pltpu.sync_copy(o_v, out_hbm.at[pl.ds(hbase, HALF)])

    return k(x, table)


def _mlp_tc(pooled, W1, b1, W2, b2):
    BB = 2048

    def body(p_ref, w1_ref, b1_ref, w2_ref, b2_ref, o_ref):
        h = jnp.dot(p_ref[...], w1_ref[...],
                    preferred_element_type=jnp.float32)
        h = jnp.maximum(h + b1_ref[...], 0.0)
        o_ref[...] = jnp.dot(h, w2_ref[...],
                             preferred_element_type=jnp.float32) + b2_ref[...]

    return pl.pallas_call(
        body,
        grid=(BATCH // BB,),
        in_specs=[
            pl.BlockSpec((BB, EMBED), lambda i: (i, 0)),
            pl.BlockSpec((EMBED, HIDDEN), lambda i: (0, 0)),
            pl.BlockSpec((1, HIDDEN), lambda i: (0, 0)),
            pl.BlockSpec((HIDDEN, CLS), lambda i: (0, 0)),
            pl.BlockSpec((1, CLS), lambda i: (0, 0)),
        ],
        out_specs=pl.BlockSpec((BB, CLS), lambda i: (i, 0)),
        out_shape=jax.ShapeDtypeStruct((BATCH, CLS), jnp.float32),
    )(pooled, W1, b1.reshape(1, HIDDEN), W2, b2.reshape(1, CLS))


def kernel(x, table, W1, b1, W2, b2):
    tail2d = table[CTILES * 128:].reshape(16, 128)
    table_rm = _transpose_sc(table.T, tail2d).reshape(VOCAB, EMBED)
    pooled = _pool_sc(x, table_rm)
    return _mlp_tc(pooled, W1, b1, W2, b2)
